# baseline (speedup 1.0000x reference)
import functools

import jax
import jax.numpy as jnp
from jax import lax
from jax.experimental import pallas as pl
from jax.experimental.pallas import tpu as pltpu
from jax.experimental.pallas import tpu_sc as plsc

D = 64
NC = 2
NS = 16
NW = NC * NS
SUB = 128
NSUB = 2
C = SUB * NSUB
L16 = 16


def _transpose_chunk(rows, tps, posrow):
    lanes = lax.iota(jnp.int32, L16)
    pj = [posrow[pl.ds(j * L16, L16)] for j in range(D // L16)]
    rowidx = [lanes + j * L16 for j in range(D // L16)]

    def body(bb, carry):
        col = lax.broadcast(bb, (L16,))
        for j in range(D // L16):
            v = rows[bb, pl.ds(j * L16, L16)] + pj[j]
            plsc.store_scatter(tps, [rowidx[j], col], v)
        return carry

    lax.fori_loop(0, C, body, 0, unroll=8)


def _emb_body(word_hbm, pos_hbm, xt_hbm, out_hbm,
              idx0, idx1, rows0, rows1, tps0, tps1, pos0, pos1,
              sem_g0, sem_g1, sem_w0, sem_w1):
    n_l, _, n_b = out_hbm.shape
    n_rows = n_l * n_b
    per_w = n_rows // NW
    chunks = per_w // C
    idx_rows_per_w = per_w // SUB
    wid = lax.axis_index("c") * NS + lax.axis_index("s")
    base = wid * per_w
    idx = (idx0, idx1)
    rows = (rows0, rows1)
    tps = (tps0, tps1)
    pos = (pos0, pos1)
    sem_g = (sem_g0, sem_g1)
    sem_w = (sem_w0, sem_w1)

    def fire(k, b):
        pltpu.sync_copy(xt_hbm.at[pl.ds(wid * idx_rows_per_w + k * NSUB,
                                        NSUB)], idx[b])
        l = (base + k * C) // n_b
        pltpu.sync_copy(pos_hbm.at[l], pos[b])
        for j in range(NSUB):
            pltpu.async_copy(word_hbm.at[idx[b].at[j]],
                             rows[b].at[pl.ds(j * SUB, SUB)], sem_g[b])

    def drain_gather(b):
        pltpu.make_async_copy(word_hbm.at[pl.ds(0, C)], rows[b],
                              sem_g[b]).wait()

    def finish(k, b):
        _transpose_chunk(rows[b], tps[b], pos[b])
        flat = base + k * C
        l = flat // n_b
        b0 = pl.multiple_of(flat - l * n_b, C)
        pltpu.async_copy(tps[b], out_hbm.at[l, :, pl.ds(b0, C)], sem_w[b])

    def wait_wb(b):
        pltpu.make_async_copy(tps[b], out_hbm.at[0, :, pl.ds(0, C)],
                              sem_w[b]).wait()

    fire(0, 0)

    def pair(t, carry):
        k0 = 2 * t
        fire(k0 + 1, 1)
        drain_gather(0)

        @pl.when(t >= 1)
        def _():
            wait_wb(0)

        finish(k0, 0)

        @pl.when(t < chunks // 2 - 1)
        def _():
            fire(k0 + 2, 0)

        drain_gather(1)

        @pl.when(t >= 1)
        def _():
            wait_wb(1)

        finish(k0 + 1, 1)
        return carry

    lax.fori_loop(0, chunks // 2, pair, 0)
    wait_wb(0)
    wait_wb(1)


@functools.partial(jax.jit, static_argnames=("n_l", "n_b"))
def _emb(word_table, pos_table, xt2d, n_l, n_b):
    mesh = plsc.VectorSubcoreMesh(core_axis_name="c", subcore_axis_name="s",
                                  num_cores=NC, num_subcores=NS)
    return pl.kernel(
        _emb_body,
        out_type=jax.ShapeDtypeStruct((n_l, D, n_b), jnp.float32),
        mesh=mesh,
        compiler_params=pltpu.CompilerParams(use_tc_tiling_on_sc=False,
                                             needs_layout_passes=False),
        scratch_types=[
            pltpu.VMEM((NSUB, SUB), jnp.int32),
            pltpu.VMEM((NSUB, SUB), jnp.int32),
            pltpu.VMEM((C, D), jnp.float32),
            pltpu.VMEM((C, D), jnp.float32),
            pltpu.VMEM((D, C), jnp.float32),
            pltpu.VMEM((D, C), jnp.float32),
            pltpu.VMEM((D,), jnp.float32),
            pltpu.VMEM((D,), jnp.float32),
            pltpu.SemaphoreType.DMA,
            pltpu.SemaphoreType.DMA,
            pltpu.SemaphoreType.DMA,
            pltpu.SemaphoreType.DMA,
        ],
    )(word_table, pos_table, xt2d)


def kernel(word_table, pos_table, x):
    Bx, Lx = x.shape
    n_rows = Bx * Lx
    xt2d = x.T.reshape(n_rows // SUB, SUB).astype(jnp.int32)
    out_t = _emb(word_table, pos_table, xt2d, Lx, Bx)
    return out_t.transpose(2, 0, 1)


# speedup vs baseline: 1.2040x; 1.2040x over previous
import functools

import jax
import jax.numpy as jnp
from jax import lax
from jax.experimental import pallas as pl
from jax.experimental.pallas import tpu as pltpu
from jax.experimental.pallas import tpu_sc as plsc

D = 64
NC = 2
NS = 16
NW = NC * NS
SUB = 128
NSUB = 2
C = SUB * NSUB
L16 = 16


def _transpose_chunk(rows, tps, posrow):
    lanes = lax.iota(jnp.int32, L16)
    pj = [posrow[pl.ds(j * L16, L16)] for j in range(D // L16)]
    rowidx = [lanes + j * L16 for j in range(D // L16)]

    @plsc.parallel_loop(0, C, unroll=8)
    def body(bb):
        col = lax.broadcast(bb, (L16,))
        for j in range(D // L16):
            v = rows[bb, pl.ds(j * L16, L16)] + pj[j]
            plsc.store_scatter(tps, [rowidx[j], col], v)


def _emb_body(word_hbm, pos_hbm, xt_hbm, out_hbm,
              idx0, idx1, rows0, rows1, tps0, tps1, pos0, pos1,
              sem_g0, sem_g1, sem_w0, sem_w1):
    n_l, _, n_b = out_hbm.shape
    n_rows = n_l * n_b
    per_w = n_rows // NW
    chunks = per_w // C
    idx_rows_per_w = per_w // SUB
    wid = lax.axis_index("c") * NS + lax.axis_index("s")
    base = wid * per_w
    idx = (idx0, idx1)
    rows = (rows0, rows1)
    tps = (tps0, tps1)
    pos = (pos0, pos1)
    sem_g = (sem_g0, sem_g1)
    sem_w = (sem_w0, sem_w1)

    def fire(k, b):
        pltpu.sync_copy(xt_hbm.at[pl.ds(wid * idx_rows_per_w + k * NSUB,
                                        NSUB)], idx[b])
        l = (base + k * C) // n_b
        pltpu.sync_copy(pos_hbm.at[l], pos[b])
        for j in range(NSUB):
            pltpu.async_copy(word_hbm.at[idx[b].at[j]],
                             rows[b].at[pl.ds(j * SUB, SUB)], sem_g[b])

    def drain_gather(b):
        pltpu.make_async_copy(word_hbm.at[pl.ds(0, C)], rows[b],
                              sem_g[b]).wait()

    def finish(k, b):
        _transpose_chunk(rows[b], tps[b], pos[b])
        flat = base + k * C
        l = flat // n_b
        b0 = pl.multiple_of(flat - l * n_b, C)
        pltpu.async_copy(tps[b], out_hbm.at[l, :, pl.ds(b0, C)], sem_w[b])

    def wait_wb(b):
        pltpu.make_async_copy(tps[b], out_hbm.at[0, :, pl.ds(0, C)],
                              sem_w[b]).wait()

    fire(0, 0)

    def pair(t, carry):
        k0 = 2 * t
        fire(k0 + 1, 1)
        drain_gather(0)

        @pl.when(t >= 1)
        def _():
            wait_wb(0)

        finish(k0, 0)

        @pl.when(t < chunks // 2 - 1)
        def _():
            fire(k0 + 2, 0)

        drain_gather(1)

        @pl.when(t >= 1)
        def _():
            wait_wb(1)

        finish(k0 + 1, 1)
        return carry

    lax.fori_loop(0, chunks // 2, pair, 0)
    wait_wb(0)
    wait_wb(1)


@functools.partial(jax.jit, static_argnames=("n_l", "n_b"))
def _emb(word_table, pos_table, xt2d, n_l, n_b):
    mesh = plsc.VectorSubcoreMesh(core_axis_name="c", subcore_axis_name="s",
                                  num_cores=NC, num_subcores=NS)
    return pl.kernel(
        _emb_body,
        out_type=jax.ShapeDtypeStruct((n_l, D, n_b), jnp.float32),
        mesh=mesh,
        compiler_params=pltpu.CompilerParams(use_tc_tiling_on_sc=False,
                                             needs_layout_passes=False),
        scratch_types=[
            pltpu.VMEM((NSUB, SUB), jnp.int32),
            pltpu.VMEM((NSUB, SUB), jnp.int32),
            pltpu.VMEM((C, D), jnp.float32),
            pltpu.VMEM((C, D), jnp.float32),
            pltpu.VMEM((D, C), jnp.float32),
            pltpu.VMEM((D, C), jnp.float32),
            pltpu.VMEM((D,), jnp.float32),
            pltpu.VMEM((D,), jnp.float32),
            pltpu.SemaphoreType.DMA,
            pltpu.SemaphoreType.DMA,
            pltpu.SemaphoreType.DMA,
            pltpu.SemaphoreType.DMA,
        ],
    )(word_table, pos_table, xt2d)


def kernel(word_table, pos_table, x):
    Bx, Lx = x.shape
    n_rows = Bx * Lx
    xt2d = x.T.reshape(n_rows // SUB, SUB).astype(jnp.int32)
    out_t = _emb(word_table, pos_table, xt2d, Lx, Bx)
    return out_t.transpose(2, 0, 1)


# speedup vs baseline: 1.8101x; 1.5034x over previous
import functools

import jax
import jax.numpy as jnp
from jax import lax
from jax.experimental import pallas as pl
from jax.experimental.pallas import tpu as pltpu
from jax.experimental.pallas import tpu_sc as plsc

D = 64
NC = 2
NS = 16
NW = NC * NS
SUB = 128
NSUB = 2
C = SUB * NSUB
L16 = 16


def _transpose_chunk(rows, tps, posrow):
    lanes = lax.iota(jnp.int32, L16)
    pj = [posrow[pl.ds(j * L16, L16)] for j in range(D // L16)]
    rowidx = [lanes + j * L16 for j in range(D // L16)]

    @plsc.parallel_loop(0, C, unroll=8)
    def body(bb):
        col = lax.broadcast(bb, (L16,))
        for j in range(D // L16):
            v = rows[bb, pl.ds(j * L16, L16)] + pj[j]
            plsc.store_scatter(tps, [rowidx[j], col], v)


def _emb_body(word_hbm, pos_hbm, xt_hbm, out_hbm,
              idx0, idx1, rows0, rows1, tps0, tps1, pos0, pos1,
              sem_g0, sem_g1, sem_w0, sem_w1):
    n_l, _, n_b = out_hbm.shape
    n_rows = n_l * n_b
    per_w = n_rows // NW
    chunks = per_w // C
    idx_rows_per_w = per_w // SUB
    wid = lax.axis_index("c") * NS + lax.axis_index("s")
    base = wid * per_w
    idx = (idx0, idx1)
    rows = (rows0, rows1)
    tps = (tps0, tps1)
    pos = (pos0, pos1)
    sem_g = (sem_g0, sem_g1)
    sem_w = (sem_w0, sem_w1)

    def fire(k, b):
        pltpu.sync_copy(xt_hbm.at[pl.ds(wid * idx_rows_per_w + k * NSUB,
                                        NSUB)], idx[b])
        l = (base + k * C) // n_b
        pltpu.sync_copy(pos_hbm.at[l], pos[b])
        for j in range(NSUB):
            pltpu.async_copy(word_hbm.at[idx[b].at[j]],
                             rows[b].at[pl.ds(j * SUB, SUB)], sem_g[b])

    def drain_gather(b):
        pltpu.make_async_copy(word_hbm.at[pl.ds(0, C)], rows[b],
                              sem_g[b]).wait()

    def finish(k, b):
        _transpose_chunk(rows[b], tps[b], pos[b])
        flat = base + k * C
        l = flat // n_b
        b0 = pl.multiple_of(flat - l * n_b, C)
        pltpu.async_copy(tps[b].at[:, pl.ds(0, C)],
                         out_hbm.at[l, :, pl.ds(b0, C)], sem_w[b])

    def wait_wb(b):
        pltpu.make_async_copy(tps[b].at[:, pl.ds(0, C)],
                              out_hbm.at[0, :, pl.ds(0, C)],
                              sem_w[b]).wait()

    fire(0, 0)

    def pair(t, carry):
        k0 = 2 * t
        fire(k0 + 1, 1)
        drain_gather(0)

        @pl.when(t >= 1)
        def _():
            wait_wb(0)

        finish(k0, 0)

        @pl.when(t < chunks // 2 - 1)
        def _():
            fire(k0 + 2, 0)

        drain_gather(1)

        @pl.when(t >= 1)
        def _():
            wait_wb(1)

        finish(k0 + 1, 1)
        return carry

    lax.fori_loop(0, chunks // 2, pair, 0)
    wait_wb(0)
    wait_wb(1)


@functools.partial(jax.jit, static_argnames=("n_l", "n_b"))
def _emb(word_table, pos_table, xt2d, n_l, n_b):
    mesh = plsc.VectorSubcoreMesh(core_axis_name="c", subcore_axis_name="s",
                                  num_cores=NC, num_subcores=NS)
    return pl.kernel(
        _emb_body,
        out_type=jax.ShapeDtypeStruct((n_l, D, n_b), jnp.float32),
        mesh=mesh,
        compiler_params=pltpu.CompilerParams(use_tc_tiling_on_sc=False,
                                             needs_layout_passes=False),
        scratch_types=[
            pltpu.VMEM((NSUB, SUB), jnp.int32),
            pltpu.VMEM((NSUB, SUB), jnp.int32),
            pltpu.VMEM((C, D), jnp.float32),
            pltpu.VMEM((C, D), jnp.float32),
            pltpu.VMEM((D, C + 1), jnp.float32),
            pltpu.VMEM((D, C + 1), jnp.float32),
            pltpu.VMEM((D,), jnp.float32),
            pltpu.VMEM((D,), jnp.float32),
            pltpu.SemaphoreType.DMA,
            pltpu.SemaphoreType.DMA,
            pltpu.SemaphoreType.DMA,
            pltpu.SemaphoreType.DMA,
        ],
    )(word_table, pos_table, xt2d)


def kernel(word_table, pos_table, x):
    Bx, Lx = x.shape
    n_rows = Bx * Lx
    xt2d = x.T.reshape(n_rows // SUB, SUB).astype(jnp.int32)
    out_t = _emb(word_table, pos_table, xt2d, Lx, Bx)
    return out_t.transpose(2, 0, 1)


# speedup vs baseline: 2.2430x; 1.2391x over previous
import functools

import jax
import jax.numpy as jnp
from jax import lax
from jax.experimental import pallas as pl
from jax.experimental.pallas import tpu as pltpu
from jax.experimental.pallas import tpu_sc as plsc

D = 64
NC = 2
NS = 16
NW = NC * NS
SUB = 128
NSUB = 2
C = SUB * NSUB
L16 = 16
TP = 257


def _transpose_chunk(rows, tps, posrow):
    lanes = lax.iota(jnp.int32, L16)
    pj = [posrow[pl.ds(j * L16, L16)] for j in range(D // L16)]
    dts = [(lanes + j * L16) // 8 for j in range(D // L16)]
    sbs = [(lanes + j * L16) % 8 for j in range(D // L16)]

    @plsc.parallel_loop(0, C, unroll=8)
    def body(bb):
        col = lax.broadcast(bb, (L16,))
        for j in range(D // L16):
            v = rows[bb, pl.ds(j * L16, L16)] + pj[j]
            plsc.store_scatter(tps, [dts[j], sbs[j], col], v)


def _emb_body(word_hbm, pos_hbm, xt_hbm, out_hbm,
              idx0, idx1, rows0, rows1, tps0, tps1, pos0, pos1,
              sem_g0, sem_g1, sem_w0, sem_w1):
    n_l = out_hbm.shape[0]
    n_b = out_hbm.shape[2] * 128
    n_rows = n_l * n_b
    per_w = n_rows // NW
    chunks = per_w // C
    idx_rows_per_w = per_w // SUB
    wid = lax.axis_index("c") * NS + lax.axis_index("s")
    base = wid * per_w
    idx = (idx0, idx1)
    rows = (rows0, rows1)
    tps = (tps0, tps1)
    pos = (pos0, pos1)
    sem_g = (sem_g0, sem_g1)
    sem_w = (sem_w0, sem_w1)

    def fire(k, b):
        pltpu.sync_copy(xt_hbm.at[pl.ds(wid * idx_rows_per_w + k * NSUB,
                                        NSUB)], idx[b])
        l = (base + k * C) // n_b
        pltpu.sync_copy(pos_hbm.at[l], pos[b])
        for j in range(NSUB):
            pltpu.async_copy(word_hbm.at[idx[b].at[j]],
                             rows[b].at[pl.ds(j * SUB, SUB)], sem_g[b])

    def drain_gather(b):
        pltpu.make_async_copy(word_hbm.at[pl.ds(0, C)], rows[b],
                              sem_g[b]).wait()

    def finish(k, b):
        _transpose_chunk(rows[b], tps[b], pos[b])
        flat = base + k * C
        l = flat // n_b
        bt0 = (flat - l * n_b) // 128
        for t in range(C // 128):
            pltpu.async_copy(tps[b].at[:, :, pl.ds(t * 128, 128)],
                             out_hbm.at[l, :, bt0 + t], sem_w[b])

    def wait_wb(b):
        pltpu.make_async_copy(tps[b].at[:, :, pl.ds(0, C)],
                              out_hbm.at[0, :, pl.ds(0, C // 128)],
                              sem_w[b]).wait()

    fire(0, 0)

    def pair(t, carry):
        k0 = 2 * t
        fire(k0 + 1, 1)
        drain_gather(0)

        @pl.when(t >= 1)
        def _():
            wait_wb(0)

        finish(k0, 0)

        @pl.when(t < chunks // 2 - 1)
        def _():
            fire(k0 + 2, 0)

        drain_gather(1)

        @pl.when(t >= 1)
        def _():
            wait_wb(1)

        finish(k0 + 1, 1)
        return carry

    lax.fori_loop(0, chunks // 2, pair, 0)
    wait_wb(0)
    wait_wb(1)


@functools.partial(jax.jit, static_argnames=("n_l", "n_b"))
def _emb(word_table, pos_table, xt2d, n_l, n_b):
    mesh = plsc.VectorSubcoreMesh(core_axis_name="c", subcore_axis_name="s",
                                  num_cores=NC, num_subcores=NS)
    return pl.kernel(
        _emb_body,
        out_type=jax.ShapeDtypeStruct((n_l, D // 8, n_b // 128, 8, 128),
                                      jnp.float32),
        mesh=mesh,
        compiler_params=pltpu.CompilerParams(use_tc_tiling_on_sc=False,
                                             needs_layout_passes=False),
        scratch_types=[
            pltpu.VMEM((NSUB, SUB), jnp.int32),
            pltpu.VMEM((NSUB, SUB), jnp.int32),
            pltpu.VMEM((C, D), jnp.float32),
            pltpu.VMEM((C, D), jnp.float32),
            pltpu.VMEM((D // 8, 8, TP), jnp.float32),
            pltpu.VMEM((D // 8, 8, TP), jnp.float32),
            pltpu.VMEM((D,), jnp.float32),
            pltpu.VMEM((D,), jnp.float32),
            pltpu.SemaphoreType.DMA,
            pltpu.SemaphoreType.DMA,
            pltpu.SemaphoreType.DMA,
            pltpu.SemaphoreType.DMA,
        ],
    )(word_table, pos_table, xt2d)


def kernel(word_table, pos_table, x):
    Bx, Lx = x.shape
    n_rows = Bx * Lx
    xt2d = x.T.reshape(n_rows // SUB, SUB).astype(jnp.int32)
    out5 = _emb(word_table, pos_table, xt2d, Lx, Bx)
    return out5.transpose(2, 4, 0, 1, 3).reshape(Bx, Lx, D)
